# baseline (speedup 1.0000x reference)
import jax
import jax.numpy as jnp
import numpy as np
from jax.experimental import pallas as pl
from jax.experimental.pallas import tpu as pltpu

H = 64
EPS = 1e-5
L = 64
TILE_R = 32768
TILE_LANES = TILE_R // L

_f32 = jnp.float32
_bf16 = jnp.bfloat16


def _acc_init(refs):
    @pl.when(pl.program_id(0) == 0)
    def _():
        for r in refs:
            r[...] = jnp.zeros_like(r)


def _stats(h, sv_ref, sq_ref):
    sv_ref[...] += jnp.sum(h, axis=0, keepdims=True)
    sq_ref[...] += jnp.sum(h * h, axis=0, keepdims=True)


def _expand_mask(m2t, e_ref):
    return jax.lax.dot_general(m2t, e_ref[...], (((0,), (0,)), ((), ())),
                               preferred_element_type=_f32)


def _dot(a, b):
    return jnp.dot(a, b, preferred_element_type=_f32)


def _p0_body(x0_ref, m_ref, w_ref, sv_ref, sq_ref, cnt_ref):
    _acc_init((sv_ref, sq_ref, cnt_ref))
    _stats(_dot(x0_ref[...], w_ref[...]), sv_ref, sq_ref)
    cnt_ref[...] += jnp.sum(m_ref[...].astype(_f32),
                            keepdims=True).reshape(1, 1)


def _mlp_body(z_ref, m_ref, wp_ref, wn_ref, a_ref, d_ref, e_ref,
              o_ref, sv_ref, sq_ref):
    _acc_init((sv_ref, sq_ref))
    h = _dot(z_ref[...], wp_ref[...])
    z = jnp.maximum(h * a_ref[...] + d_ref[...], 0.0)
    z = (z * _expand_mask(m_ref[...], e_ref)).astype(_bf16)
    o_ref[...] = z
    _stats(_dot(z, wn_ref[...]), sv_ref, sq_ref)


def _pool_cat_body(z_ref, m_ref, wp_ref, wn_ref, a_ref, d_ref, e_ref,
                   o_ref, p_ref, sv_ref, sq_ref):
    _acc_init((sv_ref, sq_ref))
    h = _dot(z_ref[...], wp_ref[...])
    mexp = _expand_mask(m_ref[...], e_ref)
    z = (jnp.maximum(h * a_ref[...] + d_ref[...], 0.0) * mexp).astype(_bf16)
    o_ref[...] = z
    pooled = jnp.max(z.reshape(TILE_LANES, L, 2 * H), axis=1)
    p_ref[...] = pooled
    pb = jnp.broadcast_to(pooled[:, None, :], (TILE_LANES, L, 2 * H))
    cat = jnp.concatenate([z, pb.reshape(TILE_R, 2 * H)], axis=-1)
    hn = _dot(cat, wn_ref[...])
    mh = hn * mexp
    sv_ref[...] += jnp.sum(mh, axis=0, keepdims=True)
    sq_ref[...] += jnp.sum(mh * hn, axis=0, keepdims=True)


def _mid0_body(z_ref, p_ref, m_ref, wp_ref, wn_ref, a_ref, d_ref, e_ref,
               o_ref, sv_ref, sq_ref):
    _acc_init((sv_ref, sq_ref))
    pb = jnp.broadcast_to(p_ref[...][:, None, :], (TILE_LANES, L, 2 * H))
    cat = jnp.concatenate([z_ref[...], pb.reshape(TILE_R, 2 * H)], axis=-1)
    h = _dot(cat, wp_ref[...])
    z = jnp.maximum(h * a_ref[...] + d_ref[...], 0.0)
    z = (z * _expand_mask(m_ref[...], e_ref)).astype(_bf16)
    o_ref[...] = z
    _stats(_dot(z, wn_ref[...]), sv_ref, sq_ref)


def _final_body(z_ref, m_ref, ml_ref, wp_ref, a_ref, d_ref, e_ref, w0_ref,
                b0_ref, w1_ref, b1_ref, y_ref):
    h = _dot(z_ref[...], wp_ref[...])
    z = jnp.maximum(h * a_ref[...] + d_ref[...], 0.0)
    z = z * _expand_mask(m_ref[...], e_ref)
    fb = jnp.max(z.reshape(TILE_LANES, L, 2 * H), axis=1)
    y = jnp.maximum(_dot(fb.astype(_bf16), w0_ref[...]) + b0_ref[...], 0.0)
    y = _dot(y.astype(_bf16), w1_ref[...]) + b1_ref[...]
    y_ref[...] = y * _expand_mask(ml_ref[...], e_ref)


def _affine(svp, sqp, cnt, g, b):
    sv = svp[0, :H] + svp[0, H:]
    sq = sqp[0, :H] + sqp[0, H:]
    mean = sv / cnt
    var = sq / cnt - mean * mean
    a = g * jax.lax.rsqrt(var + EPS)
    d = b - mean * a
    return (jnp.concatenate([a, a]).reshape(1, 2 * H),
            jnp.concatenate([d, d]).reshape(1, 2 * H))


def _bdiag(wt):
    return jnp.kron(jnp.eye(2, dtype=_f32), wt).astype(_bf16)


def kernel(lane_positions, lane_attr, lane_padding_mask, lane_key_padding_mask,
           W_pre0, g_pre0, b_pre0, W_pre1, g_pre1, b_pre1, W_pre2, g_pre2, b_pre2,
           W_mid0, g_mid0, b_mid0, W_mid1, g_mid1, b_mid1,
           W_out0, b_out0, W_out1, b_out1):
    B, M, Ll = lane_padding_mask.shape
    N = B * M * Ll
    N2 = N // 2
    grid = N2 // TILE_R

    pos = lane_positions.reshape(B * M, Ll, 2)
    vec = pos[:, 1:] - pos[:, :-1]
    vec = jnp.concatenate([jnp.zeros((B * M, 1, 2), _f32), vec], axis=1)
    valid = (~lane_padding_mask).reshape(N, 1).astype(_f32)
    vraw_x = vec[..., 0].reshape(N, 1)
    vraw_y = vec[..., 1].reshape(N, 1)
    ang = jnp.arctan2(vraw_y, vraw_x) * valid
    vx = vraw_x * valid
    vy = vraw_y * valid
    ltype = (jnp.broadcast_to(lane_attr[..., 0:1][:, :, None, :],
                              (B, M, Ll, 1)).reshape(N, 1) * valid)
    lwidth = (jnp.broadcast_to(lane_attr[..., 2:3][:, :, None, :],
                               (B, M, Ll, 1)).reshape(N, 1) * valid)
    x0 = jnp.concatenate([vx, vy, ang, ltype, lwidth], axis=1)
    x0p = jnp.concatenate([x0[:N2], x0[N2:]], axis=1).astype(_bf16)
    mp = jnp.stack([valid[:N2, 0], valid[N2:, 0]]).astype(_bf16)
    vl = (~lane_key_padding_mask).reshape(B * M, 1).astype(_f32)
    BM2 = B * M // 2
    vlp = jnp.stack([vl[:BM2, 0], vl[BM2:, 0]]).astype(_bf16)

    w0 = _bdiag(W_pre0.T)
    w1 = _bdiag(W_pre1.T)
    w2 = _bdiag(W_pre2.T)
    wcat = jnp.concatenate([_bdiag(W_mid0.T[:H]), _bdiag(W_mid0.T[H:])],
                           axis=0)
    wm1 = _bdiag(W_mid1.T)
    wo0 = _bdiag(W_out0.T)
    wo1 = _bdiag(W_out1.T)
    b0t = jnp.concatenate([b_out0, b_out0]).reshape(1, 2 * H)
    b1t = jnp.concatenate([b_out1, b_out1]).reshape(1, 2 * H)
    lane_ids = jnp.arange(2 * H) >= H
    esel = jnp.stack([(~lane_ids).astype(_bf16),
                      lane_ids.astype(_bf16)])

    row_spec = lambda c: pl.BlockSpec((TILE_R, c), lambda i: (i, 0))
    mask_spec = pl.BlockSpec((2, TILE_R), lambda i: (0, i))
    lmask_spec = pl.BlockSpec((2, TILE_LANES), lambda i: (0, i))
    lane_spec = lambda c: pl.BlockSpec((TILE_LANES, c), lambda i: (i, 0))
    full = lambda arr: pl.BlockSpec(arr.shape, lambda i: (0, 0))
    acc_spec = pl.BlockSpec((1, 2 * H), lambda i: (0, 0))
    seq = pltpu.CompilerParams(dimension_semantics=("arbitrary",))
    acc_shape = jax.ShapeDtypeStruct((1, 2 * H), _f32)
    z_shape = jax.ShapeDtypeStruct((N2, 2 * H), _bf16)

    sv1, sq1, cnt = pl.pallas_call(
        _p0_body,
        grid=(grid,),
        in_specs=[row_spec(10), mask_spec, full(w0)],
        out_specs=[acc_spec, acc_spec,
                   pl.BlockSpec((1, 1), lambda i: (0, 0))],
        out_shape=[acc_shape, acc_shape,
                   jax.ShapeDtypeStruct((1, 1), _f32)],
        compiler_params=seq,
    )(x0p, mp, w0)
    cnt = jnp.maximum(cnt[0, 0], 1.0)

    def mlp_pass(z, wp, wn, svp, sqp, g, b, zc):
        a, d = _affine(svp, sqp, cnt, g, b)
        return pl.pallas_call(
            _mlp_body,
            grid=(grid,),
            in_specs=[row_spec(zc), mask_spec, full(wp), full(wn), full(a),
                      full(d), full(esel)],
            out_specs=[row_spec(2 * H), acc_spec, acc_spec],
            out_shape=[z_shape, acc_shape, acc_shape],
            compiler_params=seq,
        )(z, mp, wp, wn, a, d, esel)

    z1, sv2, sq2 = mlp_pass(x0p, w0, w1, sv1, sq1, g_pre0, b_pre0, 10)
    z2, sv3, sq3 = mlp_pass(z1, w1, w2, sv2, sq2, g_pre1, b_pre1, 2 * H)

    a3, d3 = _affine(sv3, sq3, cnt, g_pre2, b_pre2)
    z3, pooled, sv4, sq4 = pl.pallas_call(
        _pool_cat_body,
        grid=(grid,),
        in_specs=[row_spec(2 * H), mask_spec, full(w2), full(wcat),
                  full(a3), full(d3), full(esel)],
        out_specs=[row_spec(2 * H), lane_spec(2 * H), acc_spec, acc_spec],
        out_shape=[z_shape, jax.ShapeDtypeStruct((BM2, 2 * H), _bf16),
                   acc_shape, acc_shape],
        compiler_params=seq,
    )(z2, mp, w2, wcat, a3, d3, esel)

    a4, d4 = _affine(sv4, sq4, cnt, g_mid0, b_mid0)
    z4, sv5, sq5 = pl.pallas_call(
        _mid0_body,
        grid=(grid,),
        in_specs=[row_spec(2 * H), lane_spec(2 * H), mask_spec, full(wcat),
                  full(wm1), full(a4), full(d4), full(esel)],
        out_specs=[row_spec(2 * H), acc_spec, acc_spec],
        out_shape=[z_shape, acc_shape, acc_shape],
        compiler_params=seq,
    )(z3, pooled, mp, wcat, wm1, a4, d4, esel)

    a5, d5 = _affine(sv5, sq5, cnt, g_mid1, b_mid1)
    y = pl.pallas_call(
        _final_body,
        grid=(grid,),
        in_specs=[row_spec(2 * H), mask_spec, lmask_spec, full(wm1),
                  full(a5), full(d5), full(esel), full(wo0), full(b0t),
                  full(wo1), full(b1t)],
        out_specs=lane_spec(2 * H),
        out_shape=jax.ShapeDtypeStruct((BM2, 2 * H), _f32),
        compiler_params=seq,
    )(z4, mp, vlp, wm1, a5, d5, esel, wo0, b0t, wo1, b1t)

    y = jnp.concatenate([y[:, :H], y[:, H:]], axis=0)
    return y.reshape(B, M, H)


# speedup vs baseline: 1.0134x; 1.0134x over previous
import jax
import jax.numpy as jnp
import numpy as np
from jax.experimental import pallas as pl
from jax.experimental.pallas import tpu as pltpu

H = 64
EPS = 1e-5
L = 64
TILE_R = 16384
TILE_LANES = TILE_R // L

_f32 = jnp.float32
_bf16 = jnp.bfloat16


def _acc_init(refs):
    @pl.when(pl.program_id(0) == 0)
    def _():
        for r in refs:
            r[...] = jnp.zeros_like(r)


def _stats(h, sv_ref, sq_ref):
    sv_ref[...] += jnp.sum(h, axis=0, keepdims=True)
    sq_ref[...] += jnp.sum(h * h, axis=0, keepdims=True)


def _expand_mask(m2t, e_ref):
    return jax.lax.dot_general(m2t, e_ref[...], (((0,), (0,)), ((), ())),
                               preferred_element_type=_f32)


def _dot(a, b):
    return jnp.dot(a, b, preferred_element_type=_f32)


def _p0_body(x0_ref, m_ref, w_ref, sv_ref, sq_ref, cnt_ref):
    _acc_init((sv_ref, sq_ref, cnt_ref))
    _stats(_dot(x0_ref[...], w_ref[...]), sv_ref, sq_ref)
    cnt_ref[...] += jnp.sum(m_ref[...].astype(_f32),
                            keepdims=True).reshape(1, 1)


def _mlp_body(z_ref, m_ref, wp_ref, wn_ref, a_ref, d_ref, e_ref,
              o_ref, sv_ref, sq_ref):
    _acc_init((sv_ref, sq_ref))
    h = _dot(z_ref[...], wp_ref[...])
    z = jnp.maximum(h * a_ref[...] + d_ref[...], 0.0)
    z = (z * _expand_mask(m_ref[...], e_ref)).astype(_bf16)
    o_ref[...] = z
    _stats(_dot(z, wn_ref[...]), sv_ref, sq_ref)


def _pool_cat_body(z_ref, m_ref, wp_ref, wn_ref, a_ref, d_ref, e_ref,
                   o_ref, p_ref, sv_ref, sq_ref):
    _acc_init((sv_ref, sq_ref))
    h = _dot(z_ref[...], wp_ref[...])
    mexp = _expand_mask(m_ref[...], e_ref)
    z = (jnp.maximum(h * a_ref[...] + d_ref[...], 0.0) * mexp).astype(_bf16)
    o_ref[...] = z
    pooled = jnp.max(z.reshape(TILE_LANES, L, 2 * H), axis=1)
    p_ref[...] = pooled
    pb = jnp.broadcast_to(pooled[:, None, :], (TILE_LANES, L, 2 * H))
    cat = jnp.concatenate([z, pb.reshape(TILE_R, 2 * H)], axis=-1)
    hn = _dot(cat, wn_ref[...])
    mh = hn * mexp
    sv_ref[...] += jnp.sum(mh, axis=0, keepdims=True)
    sq_ref[...] += jnp.sum(mh * hn, axis=0, keepdims=True)


def _mid0_body(z_ref, p_ref, m_ref, wp_ref, wn_ref, a_ref, d_ref, e_ref,
               o_ref, sv_ref, sq_ref):
    _acc_init((sv_ref, sq_ref))
    pb = jnp.broadcast_to(p_ref[...][:, None, :], (TILE_LANES, L, 2 * H))
    cat = jnp.concatenate([z_ref[...], pb.reshape(TILE_R, 2 * H)], axis=-1)
    h = _dot(cat, wp_ref[...])
    z = jnp.maximum(h * a_ref[...] + d_ref[...], 0.0)
    z = (z * _expand_mask(m_ref[...], e_ref)).astype(_bf16)
    o_ref[...] = z
    _stats(_dot(z, wn_ref[...]), sv_ref, sq_ref)


def _final_body(z_ref, m_ref, ml_ref, wp_ref, a_ref, d_ref, e_ref, w0_ref,
                b0_ref, w1_ref, b1_ref, y_ref):
    h = _dot(z_ref[...], wp_ref[...])
    z = jnp.maximum(h * a_ref[...] + d_ref[...], 0.0)
    z = z * _expand_mask(m_ref[...], e_ref)
    fb = jnp.max(z.reshape(TILE_LANES, L, 2 * H), axis=1)
    y = jnp.maximum(_dot(fb.astype(_bf16), w0_ref[...]) + b0_ref[...], 0.0)
    y = _dot(y.astype(_bf16), w1_ref[...]) + b1_ref[...]
    y_ref[...] = y * _expand_mask(ml_ref[...], e_ref)


def _affine(svp, sqp, cnt, g, b):
    sv = svp[0, :H] + svp[0, H:]
    sq = sqp[0, :H] + sqp[0, H:]
    mean = sv / cnt
    var = sq / cnt - mean * mean
    a = g * jax.lax.rsqrt(var + EPS)
    d = b - mean * a
    return (jnp.concatenate([a, a]).reshape(1, 2 * H),
            jnp.concatenate([d, d]).reshape(1, 2 * H))


def _bdiag(wt):
    return jnp.kron(jnp.eye(2, dtype=_f32), wt).astype(_bf16)


def kernel(lane_positions, lane_attr, lane_padding_mask, lane_key_padding_mask,
           W_pre0, g_pre0, b_pre0, W_pre1, g_pre1, b_pre1, W_pre2, g_pre2, b_pre2,
           W_mid0, g_mid0, b_mid0, W_mid1, g_mid1, b_mid1,
           W_out0, b_out0, W_out1, b_out1):
    B, M, Ll = lane_padding_mask.shape
    N = B * M * Ll
    N2 = N // 2
    grid = N2 // TILE_R

    pos = lane_positions.reshape(B * M, Ll, 2)
    vec = pos[:, 1:] - pos[:, :-1]
    vec = jnp.concatenate([jnp.zeros((B * M, 1, 2), _f32), vec], axis=1)
    valid = (~lane_padding_mask).reshape(N, 1).astype(_f32)
    vraw_x = vec[..., 0].reshape(N, 1)
    vraw_y = vec[..., 1].reshape(N, 1)
    ang = jnp.arctan2(vraw_y, vraw_x) * valid
    vx = vraw_x * valid
    vy = vraw_y * valid
    ltype = (jnp.broadcast_to(lane_attr[..., 0:1][:, :, None, :],
                              (B, M, Ll, 1)).reshape(N, 1) * valid)
    lwidth = (jnp.broadcast_to(lane_attr[..., 2:3][:, :, None, :],
                               (B, M, Ll, 1)).reshape(N, 1) * valid)
    x0 = jnp.concatenate([vx, vy, ang, ltype, lwidth], axis=1)
    x0p = jnp.concatenate([x0[:N2], x0[N2:]], axis=1).astype(_bf16)
    mp = jnp.stack([valid[:N2, 0], valid[N2:, 0]]).astype(_bf16)
    vl = (~lane_key_padding_mask).reshape(B * M, 1).astype(_f32)
    BM2 = B * M // 2
    vlp = jnp.stack([vl[:BM2, 0], vl[BM2:, 0]]).astype(_bf16)

    w0 = _bdiag(W_pre0.T)
    w1 = _bdiag(W_pre1.T)
    w2 = _bdiag(W_pre2.T)
    wcat = jnp.concatenate([_bdiag(W_mid0.T[:H]), _bdiag(W_mid0.T[H:])],
                           axis=0)
    wm1 = _bdiag(W_mid1.T)
    wo0 = _bdiag(W_out0.T)
    wo1 = _bdiag(W_out1.T)
    b0t = jnp.concatenate([b_out0, b_out0]).reshape(1, 2 * H)
    b1t = jnp.concatenate([b_out1, b_out1]).reshape(1, 2 * H)
    lane_ids = jnp.arange(2 * H) >= H
    esel = jnp.stack([(~lane_ids).astype(_bf16),
                      lane_ids.astype(_bf16)])

    row_spec = lambda c: pl.BlockSpec((TILE_R, c), lambda i: (i, 0))
    mask_spec = pl.BlockSpec((2, TILE_R), lambda i: (0, i))
    lmask_spec = pl.BlockSpec((2, TILE_LANES), lambda i: (0, i))
    lane_spec = lambda c: pl.BlockSpec((TILE_LANES, c), lambda i: (i, 0))
    full = lambda arr: pl.BlockSpec(arr.shape, lambda i: (0, 0))
    acc_spec = pl.BlockSpec((1, 2 * H), lambda i: (0, 0))
    seq = pltpu.CompilerParams(dimension_semantics=("arbitrary",))
    acc_shape = jax.ShapeDtypeStruct((1, 2 * H), _f32)
    z_shape = jax.ShapeDtypeStruct((N2, 2 * H), _bf16)

    sv1, sq1, cnt = pl.pallas_call(
        _p0_body,
        grid=(grid,),
        in_specs=[row_spec(10), mask_spec, full(w0)],
        out_specs=[acc_spec, acc_spec,
                   pl.BlockSpec((1, 1), lambda i: (0, 0))],
        out_shape=[acc_shape, acc_shape,
                   jax.ShapeDtypeStruct((1, 1), _f32)],
        compiler_params=seq,
    )(x0p, mp, w0)
    cnt = jnp.maximum(cnt[0, 0], 1.0)

    def mlp_pass(z, wp, wn, svp, sqp, g, b, zc):
        a, d = _affine(svp, sqp, cnt, g, b)
        return pl.pallas_call(
            _mlp_body,
            grid=(grid,),
            in_specs=[row_spec(zc), mask_spec, full(wp), full(wn), full(a),
                      full(d), full(esel)],
            out_specs=[row_spec(2 * H), acc_spec, acc_spec],
            out_shape=[z_shape, acc_shape, acc_shape],
            compiler_params=seq,
        )(z, mp, wp, wn, a, d, esel)

    z1, sv2, sq2 = mlp_pass(x0p, w0, w1, sv1, sq1, g_pre0, b_pre0, 10)
    z2, sv3, sq3 = mlp_pass(z1, w1, w2, sv2, sq2, g_pre1, b_pre1, 2 * H)

    a3, d3 = _affine(sv3, sq3, cnt, g_pre2, b_pre2)
    z3, pooled, sv4, sq4 = pl.pallas_call(
        _pool_cat_body,
        grid=(grid,),
        in_specs=[row_spec(2 * H), mask_spec, full(w2), full(wcat),
                  full(a3), full(d3), full(esel)],
        out_specs=[row_spec(2 * H), lane_spec(2 * H), acc_spec, acc_spec],
        out_shape=[z_shape, jax.ShapeDtypeStruct((BM2, 2 * H), _bf16),
                   acc_shape, acc_shape],
        compiler_params=seq,
    )(z2, mp, w2, wcat, a3, d3, esel)

    a4, d4 = _affine(sv4, sq4, cnt, g_mid0, b_mid0)
    z4, sv5, sq5 = pl.pallas_call(
        _mid0_body,
        grid=(grid,),
        in_specs=[row_spec(2 * H), lane_spec(2 * H), mask_spec, full(wcat),
                  full(wm1), full(a4), full(d4), full(esel)],
        out_specs=[row_spec(2 * H), acc_spec, acc_spec],
        out_shape=[z_shape, acc_shape, acc_shape],
        compiler_params=seq,
    )(z3, pooled, mp, wcat, wm1, a4, d4, esel)

    a5, d5 = _affine(sv5, sq5, cnt, g_mid1, b_mid1)
    y = pl.pallas_call(
        _final_body,
        grid=(grid,),
        in_specs=[row_spec(2 * H), mask_spec, lmask_spec, full(wm1),
                  full(a5), full(d5), full(esel), full(wo0), full(b0t),
                  full(wo1), full(b1t)],
        out_specs=lane_spec(2 * H),
        out_shape=jax.ShapeDtypeStruct((BM2, 2 * H), _f32),
        compiler_params=seq,
    )(z4, mp, vlp, wm1, a5, d5, esel, wo0, b0t, wo1, b1t)

    y = jnp.concatenate([y[:, :H], y[:, H:]], axis=0)
    return y.reshape(B, M, H)
